# baseline (speedup 1.0000x reference)
import jax
import jax.numpy as jnp
from jax.experimental import pallas as pl
from jax.experimental.pallas import tpu as pltpu


def kernel(x, pe_weight):
    seq_len = x.shape[1]
    n_rows, dim = pe_weight.shape
    del n_rows

    n_buf = 4
    taper = [256, 256, 512, 1024]
    chunks = []
    pos = 0
    for c in taper:
        chunks.append((pos, c))
        pos += c
    tail = list(reversed(taper))
    mid = seq_len - pos - sum(tail)
    while mid > 0:
        c = min(2048, mid)
        chunks.append((pos, c))
        pos += c
        mid -= c
    for c in tail:
        chunks.append((pos, c))
        pos += c
    if pos != seq_len:
        chunks = [(o, 512) for o in range(0, seq_len, 512)]
    max_rows = max(c for _, c in chunks)
    n_chunks = len(chunks)

    def copy_body(src_hbm, out_hbm, bufs, in_sems, out_sems):
        def start_in(i):
            b = i % n_buf
            off, rows = chunks[i]
            pltpu.make_async_copy(
                src_hbm.at[pl.ds(off, rows)],
                bufs.at[b, pl.ds(0, rows)],
                in_sems.at[b],
            ).start()

        def wait_in(i):
            b = i % n_buf
            off, rows = chunks[i]
            pltpu.make_async_copy(
                src_hbm.at[pl.ds(off, rows)],
                bufs.at[b, pl.ds(0, rows)],
                in_sems.at[b],
            ).wait()

        def start_out(i):
            b = i % n_buf
            off, rows = chunks[i]
            pltpu.make_async_copy(
                bufs.at[b, pl.ds(0, rows)],
                out_hbm.at[pl.ds(off, rows)],
                out_sems.at[b],
            ).start()

        def wait_out(i):
            b = i % n_buf
            off, rows = chunks[i]
            pltpu.make_async_copy(
                bufs.at[b, pl.ds(0, rows)],
                out_hbm.at[pl.ds(off, rows)],
                out_sems.at[b],
            ).wait()

        for i in range(min(n_buf, n_chunks)):
            start_in(i)
        for i in range(n_chunks):
            wait_in(i)
            start_out(i)
            if i + n_buf < n_chunks:
                wait_out(i)
                start_in(i + n_buf)
        for i in range(max(n_chunks - n_buf, 0), n_chunks):
            wait_out(i)

    out = pl.pallas_call(
        copy_body,
        out_shape=jax.ShapeDtypeStruct((seq_len, dim), pe_weight.dtype),
        in_specs=[pl.BlockSpec(memory_space=pltpu.MemorySpace.HBM)],
        out_specs=pl.BlockSpec(memory_space=pltpu.MemorySpace.HBM),
        scratch_shapes=[
            pltpu.VMEM((n_buf, max_rows, dim), pe_weight.dtype),
            pltpu.SemaphoreType.DMA((n_buf,)),
            pltpu.SemaphoreType.DMA((n_buf,)),
        ],
    )(pe_weight)
    return out[None, ...]
